# baseline (speedup 1.0000x reference)
import jax
import jax.numpy as jnp
from jax import lax
from jax.experimental import pallas as pl
from jax.experimental.pallas import tpu as pltpu
from jax.experimental.pallas import tpu_sc as plsc

NC, NS = 2, 16
NW = NC * NS
B, L, DIM = 16384, 50, 32
NUM_ROWS = 1000000
BAGS_W = B // NW
CB = 32
NCH = BAGS_W // CB
NBUF = 2
RPC = CB * L
INV_L = 1.0 / L


def _body(tok_hbm, tab_hbm, out_hbm, *scratch):
    wid = lax.axis_index("s") * NC + lax.axis_index("c")
    idx = scratch[0:NBUF]
    rows = scratch[NBUF:2 * NBUF]
    outs = scratch[2 * NBUF:3 * NBUF]
    sems = scratch[3 * NBUF:4 * NBUF]

    def fire(c, p):
        bag0 = wid * BAGS_W + c * CB
        pltpu.sync_copy(tok_hbm.at[pl.ds(bag0 * L, RPC)], idx[p])
        pltpu.async_copy(tab_hbm.at[idx[p]], rows[p], sems[p])

    def process(c, p):
        pltpu.make_async_copy(tab_hbm.at[idx[p]], rows[p], sems[p]).wait()
        rv = rows[p]
        ov = outs[p]

        def bag(b, carry):
            a0, a1 = plsc.unpack(rv[b * L, 0:DIM],
                                 format=plsc.PackFormat.INTERLEAVED)
            for r in range(1, L):
                e, o = plsc.unpack(rv[b * L + r, 0:DIM],
                                   format=plsc.PackFormat.INTERLEAVED)
                a0 = a0 + e
                a1 = a1 + o
            ov[b, 0:16] = a0 * INV_L
            ov[b, 16:32] = a1 * INV_L
            return carry

        lax.fori_loop(0, CB, bag, 0)
        bag0 = wid * BAGS_W + c * CB
        pltpu.sync_copy(ov, out_hbm.at[pl.ds(bag0, CB)])

    for p in range(NBUF):
        fire(p, p)

    def step(s, carry):
        c0 = s * NBUF
        for p in range(NBUF):
            process(c0 + p, p)

            @pl.when(c0 + p + NBUF < NCH)
            def _():
                fire(c0 + p + NBUF, p)

        return carry

    lax.fori_loop(0, NCH // NBUF, step, 0)


TBLK = 65536
TGRID = -(-NUM_ROWS // TBLK)
TROWS = TBLK * DIM // 128


def _transpose_body(x_ref, o_ref):
    x = x_ref[...]
    xx = jnp.concatenate(
        [x[:, TROWS * j:TROWS * (j + 1)] for j in range(4)], axis=0)
    o_ref[...] = jnp.transpose(xx).astype(jnp.bfloat16)


def _linearize_table(emb_weight):
    tab_t = emb_weight.T
    out = pl.pallas_call(
        _transpose_body,
        grid=(TGRID,),
        in_specs=[pl.BlockSpec((DIM, TBLK), lambda g: (0, g))],
        out_specs=pl.BlockSpec((TROWS, 128), lambda g: (g, 0)),
        out_shape=jax.ShapeDtypeStruct((TGRID * TROWS, 128), jnp.bfloat16),
    )(tab_t)
    return out.reshape(TGRID * TBLK, DIM)


def _permute_tokens(tok):
    g = tok // TBLK
    w = tok % TBLK
    return (g * TROWS + w % TROWS) * 4 + w // TROWS


def kernel(tokens_idx, emb_weight):
    tok = _permute_tokens(tokens_idx.reshape(-1).astype(jnp.int32))
    tab = _linearize_table(emb_weight)
    mesh = plsc.VectorSubcoreMesh(core_axis_name="c", subcore_axis_name="s")
    f = pl.kernel(
        _body,
        out_type=jax.ShapeDtypeStruct((B, DIM), jnp.float32),
        mesh=mesh,
        compiler_params=pltpu.CompilerParams(use_tc_tiling_on_sc=False,
                                             needs_layout_passes=False),
        scratch_types=(
            [pltpu.VMEM((RPC,), jnp.int32)] * NBUF
            + [pltpu.VMEM((RPC, DIM), jnp.bfloat16)] * NBUF
            + [pltpu.VMEM((CB, DIM), jnp.float32)] * NBUF
            + [pltpu.SemaphoreType.DMA] * NBUF
        ),
    )
    out = f(tok, tab)
    return out.reshape(B, 2, 16).transpose(0, 2, 1).reshape(B, DIM)


# speedup vs baseline: 2.0037x; 2.0037x over previous
import jax
import jax.numpy as jnp
from jax import lax
from jax.experimental import pallas as pl
from jax.experimental.pallas import tpu as pltpu
from jax.experimental.pallas import tpu_sc as plsc

NC, NS = 2, 16
NW = NC * NS
B, L, DIM = 16384, 50, 32
NUM_ROWS = 1000000
BAGS_W = B // NW
CB = 32
NCH = BAGS_W // CB
NBUF = 2
RPC = CB * L
INV_L = 1.0 / L


def _body(tok_hbm, tab_hbm, out_hbm, *scratch):
    wid = lax.axis_index("s") * NC + lax.axis_index("c")
    idx = scratch[0:NBUF]
    rows = scratch[NBUF:2 * NBUF]
    outs = scratch[2 * NBUF:3 * NBUF]
    sems = scratch[3 * NBUF:4 * NBUF]

    def fire(c, p):
        bag0 = wid * BAGS_W + c * CB
        pltpu.sync_copy(tok_hbm.at[pl.ds(bag0 * L, RPC)], idx[p])
        pltpu.async_copy(tab_hbm.at[idx[p]], rows[p], sems[p])

    def process(c, p):
        pltpu.make_async_copy(tab_hbm.at[idx[p]], rows[p], sems[p]).wait()
        rv = rows[p]
        ov = outs[p]

        def bag(b, carry):
            a0 = rv[b * L, 0:16]
            a1 = rv[b * L, 16:32]
            for r in range(1, L):
                a0 = a0 + rv[b * L + r, 0:16]
                a1 = a1 + rv[b * L + r, 16:32]
            ov[b, 0:16] = a0 * INV_L
            ov[b, 16:32] = a1 * INV_L
            return carry

        lax.fori_loop(0, CB, bag, 0)
        bag0 = wid * BAGS_W + c * CB
        pltpu.sync_copy(ov, out_hbm.at[pl.ds(bag0, CB)])

    for p in range(NBUF):
        fire(p, p)

    def step(s, carry):
        c0 = s * NBUF
        for p in range(NBUF):
            process(c0 + p, p)

            @pl.when(c0 + p + NBUF < NCH)
            def _():
                fire(c0 + p + NBUF, p)

        return carry

    lax.fori_loop(0, NCH // NBUF, step, 0)


TBLK = 65536
TGRID = -(-NUM_ROWS // TBLK)
TROWS = TBLK * DIM // 128


def _transpose_body(x_ref, o_ref):
    x = x_ref[...]
    xx = jnp.concatenate(
        [x[:, TROWS * j:TROWS * (j + 1)] for j in range(4)], axis=0)
    o_ref[...] = jnp.transpose(xx)


def _linearize_table(emb_weight):
    tab_t = emb_weight.T
    out = pl.pallas_call(
        _transpose_body,
        grid=(TGRID,),
        in_specs=[pl.BlockSpec((DIM, TBLK), lambda g: (0, g))],
        out_specs=pl.BlockSpec((TROWS, 128), lambda g: (g, 0)),
        out_shape=jax.ShapeDtypeStruct((TGRID * TROWS, 128), jnp.float32),
    )(tab_t)
    return out.reshape(TGRID * TBLK, DIM)


def _permute_tokens(tok):
    g = tok // TBLK
    w = tok % TBLK
    return (g * TROWS + w % TROWS) * 4 + w // TROWS


def kernel(tokens_idx, emb_weight):
    tok = _permute_tokens(tokens_idx.reshape(-1).astype(jnp.int32))
    tab = _linearize_table(emb_weight)
    mesh = plsc.VectorSubcoreMesh(core_axis_name="c", subcore_axis_name="s")
    f = pl.kernel(
        _body,
        out_type=jax.ShapeDtypeStruct((B, DIM), jnp.float32),
        mesh=mesh,
        compiler_params=pltpu.CompilerParams(use_tc_tiling_on_sc=False),
        scratch_types=(
            [pltpu.VMEM((RPC,), jnp.int32)] * NBUF
            + [pltpu.VMEM((RPC, DIM), jnp.float32)] * NBUF
            + [pltpu.VMEM((CB, DIM), jnp.float32)] * NBUF
            + [pltpu.SemaphoreType.DMA] * NBUF
        ),
    )
    return f(tok, tab)
